# baseline (speedup 1.0000x reference)
import functools

import jax
import jax.numpy as jnp
from jax import lax
from jax.experimental import pallas as pl
from jax.experimental.pallas import tpu as pltpu
from jax.experimental.pallas import tpu_sc as plsc

_NC = 2
_NS = 16
_NW = _NC * _NS


@functools.partial(jax.jit, static_argnames=("n_chunks", "chunk"))
def _sc_gather(table, idx3, *, n_chunks, chunk):
    n_rows = _NW * n_chunks * chunk
    d = table.shape[1]
    mesh = plsc.VectorSubcoreMesh(core_axis_name="c", subcore_axis_name="s")

    @functools.partial(
        pl.kernel,
        mesh=mesh,
        out_type=jax.ShapeDtypeStruct((n_rows, d), jnp.float32),
        scratch_types=[
            pltpu.VMEM((chunk,), jnp.int32),
            pltpu.VMEM((chunk,), jnp.int32),
            pltpu.VMEM((chunk, d), jnp.float32),
            pltpu.VMEM((chunk, d), jnp.float32),
            pltpu.SemaphoreType.DMA,
            pltpu.SemaphoreType.DMA,
        ],
        compiler_params=pltpu.CompilerParams(use_tc_tiling_on_sc=False),
    )
    def k(table_hbm, idx_hbm, out_hbm, idx0, idx1, buf0, buf1, sem0, sem1):
        wid = lax.axis_index("s") * _NC + lax.axis_index("c")
        base = wid * (n_chunks * chunk)
        idxs = (idx0, idx1)
        bufs = (buf0, buf1)
        sems = (sem0, sem1)
        cps = [None, None]
        pltpu.sync_copy(idx_hbm.at[wid, 0], idxs[0])
        cps[0] = pltpu.async_copy(table_hbm.at[idxs[0]], bufs[0], sems[0])
        for i in range(n_chunks):
            cur = i % 2
            if i + 1 < n_chunks:
                nxt = (i + 1) % 2
                pltpu.sync_copy(idx_hbm.at[wid, i + 1], idxs[nxt])
                cps[nxt] = pltpu.async_copy(
                    table_hbm.at[idxs[nxt]], bufs[nxt], sems[nxt])
            cps[cur].wait()
            pltpu.sync_copy(bufs[cur], out_hbm.at[pl.ds(base + i * chunk, chunk)])

    return k(table, idx3)


def _mm_body(emb_ref, wt_ref, b_ref, out_ref):
    out_ref[...] = (
        jnp.dot(emb_ref[...], wt_ref[...], preferred_element_type=jnp.float32)
        + b_ref[...]
    )


@functools.partial(jax.jit, static_argnames=("block_rows",))
def _tc_project(emb, wt, b2, *, block_rows):
    n, d = emb.shape
    h = wt.shape[1]
    grid = (n // block_rows,)
    return pl.pallas_call(
        _mm_body,
        grid=grid,
        in_specs=[
            pl.BlockSpec((block_rows, d), lambda i: (i, 0)),
            pl.BlockSpec((d, h), lambda i: (0, 0)),
            pl.BlockSpec((1, h), lambda i: (0, 0)),
        ],
        out_specs=pl.BlockSpec((block_rows, h), lambda i: (i, 0)),
        out_shape=jax.ShapeDtypeStruct((n, h), jnp.float32),
    )(emb, wt, b2)


def kernel(x, table, W, b):
    bsz, seq = x.shape
    n_rows = bsz * seq
    chunk = 640
    per_w = n_rows // _NW
    n_chunks = per_w // chunk
    idx3 = x.reshape(_NW, n_chunks, chunk).astype(jnp.int32)
    emb = _sc_gather(table, idx3, n_chunks=n_chunks, chunk=chunk)
    out = _tc_project(emb, W.T, b.reshape(1, -1), block_rows=1024)
    return out.reshape(bsz, seq, -1)


# speedup vs baseline: 1.3656x; 1.3656x over previous
import functools

import jax
import jax.numpy as jnp
from jax import lax
from jax.experimental import pallas as pl
from jax.experimental.pallas import tpu as pltpu
from jax.experimental.pallas import tpu_sc as plsc

_NC = 2
_NS = 16
_NW = _NC * _NS


@functools.partial(jax.jit, static_argnames=("n_chunks", "chunk"))
def _sc_gather(table, idx3, *, n_chunks, chunk):
    n_rows = _NW * n_chunks * chunk
    d = table.shape[1]
    mesh = plsc.VectorSubcoreMesh(core_axis_name="c", subcore_axis_name="s")

    @functools.partial(
        pl.kernel,
        mesh=mesh,
        out_type=jax.ShapeDtypeStruct((n_rows, 128), jnp.float32),
        scratch_types=[
            pltpu.VMEM((chunk,), jnp.int32),
            pltpu.VMEM((chunk,), jnp.int32),
            pltpu.VMEM((chunk, d), jnp.float32),
            pltpu.VMEM((chunk, d), jnp.float32),
            pltpu.SemaphoreType.DMA,
            pltpu.SemaphoreType.DMA,
        ],
        compiler_params=pltpu.CompilerParams(use_tc_tiling_on_sc=False),
    )
    def k(table_hbm, idx_hbm, out_hbm, idx0, idx1, buf0, buf1, sem0, sem1):
        wid = lax.axis_index("s") * _NC + lax.axis_index("c")
        base = wid * (n_chunks * chunk)
        idxs = (idx0, idx1)
        bufs = (buf0, buf1)
        sems = (sem0, sem1)
        cps = [None, None]
        pltpu.sync_copy(idx_hbm.at[wid, 0], idxs[0])
        cps[0] = pltpu.async_copy(table_hbm.at[idxs[0]], bufs[0], sems[0])
        for i in range(n_chunks):
            cur = i % 2
            if i + 1 < n_chunks:
                nxt = (i + 1) % 2
                pltpu.sync_copy(idx_hbm.at[wid, i + 1], idxs[nxt])
                cps[nxt] = pltpu.async_copy(
                    table_hbm.at[idxs[nxt]], bufs[nxt], sems[nxt])
            cps[cur].wait()
            pltpu.sync_copy(
                bufs[cur],
                out_hbm.at[pl.ds(base + i * chunk, chunk), pl.ds(0, d)])

    return k(table, idx3)


def _mm_body(emb_ref, wt_ref, b_ref, out_ref):
    bm, ll, h = out_ref.shape
    e = emb_ref[:, :64]
    r = jnp.dot(e, wt_ref[...], preferred_element_type=jnp.float32) + b_ref[...]
    out_ref[...] = r.reshape(bm, ll, h)


@functools.partial(jax.jit, static_argnames=("bm", "seq"))
def _tc_project(emb, wt, b2, *, bm, seq):
    d, h = wt.shape
    n = emb.shape[0]
    bsz = n // seq
    grid = (bsz // bm,)
    return pl.pallas_call(
        _mm_body,
        grid=grid,
        in_specs=[
            pl.BlockSpec((bm * seq, 128), lambda i: (i, 0)),
            pl.BlockSpec((d, h), lambda i: (0, 0)),
            pl.BlockSpec((1, h), lambda i: (0, 0)),
        ],
        out_specs=pl.BlockSpec((bm, seq, h), lambda i: (i, 0, 0)),
        out_shape=jax.ShapeDtypeStruct((bsz, seq, h), jnp.float32),
    )(emb, wt, b2)


def kernel(x, table, W, b):
    bsz, seq = x.shape
    n_rows = bsz * seq
    chunk = 640
    per_w = n_rows // _NW
    n_chunks = per_w // chunk
    idx3 = x.reshape(_NW, n_chunks, chunk).astype(jnp.int32)
    emb = _sc_gather(table, idx3, n_chunks=n_chunks, chunk=chunk)
    out = _tc_project(emb, W.T, b.reshape(1, -1), bm=32, seq=seq)
    return out


# speedup vs baseline: 1.9936x; 1.4599x over previous
import functools

import jax
import jax.numpy as jnp
from jax import lax
from jax.experimental import pallas as pl
from jax.experimental.pallas import tpu as pltpu
from jax.experimental.pallas import tpu_sc as plsc

_NC = 2
_NS = 16
_NW = _NC * _NS


@functools.partial(jax.jit, static_argnames=("n_chunks", "chunk"))
def _sc_gather(table, idx3, *, n_chunks, chunk):
    n_rows = _NW * n_chunks * chunk
    d = table.shape[1]
    mesh = plsc.VectorSubcoreMesh(core_axis_name="c", subcore_axis_name="s")

    @functools.partial(
        pl.kernel,
        mesh=mesh,
        out_type=jax.ShapeDtypeStruct((n_rows, 128), jnp.float32),
        scratch_types=[
            pltpu.VMEM((chunk,), jnp.int32),
            pltpu.VMEM((chunk,), jnp.int32),
            pltpu.VMEM((chunk, d), jnp.float32),
            pltpu.VMEM((chunk, d), jnp.float32),
            pltpu.SemaphoreType.DMA,
            pltpu.SemaphoreType.DMA,
        ],
        compiler_params=pltpu.CompilerParams(use_tc_tiling_on_sc=False),
    )
    def k(table_hbm, idx_hbm, out_hbm, idx0, idx1, buf0, buf1, sem0, sem1):
        wid = lax.axis_index("s") * _NC + lax.axis_index("c")
        base = wid * (n_chunks * chunk)
        idxs = (idx0, idx1)
        bufs = (buf0, buf1)
        sems = (sem0, sem1)
        cps = [None, None]
        pltpu.sync_copy(idx_hbm.at[wid, 0], idxs[0])
        cps[0] = pltpu.async_copy(table_hbm.at[idxs[0]], bufs[0], sems[0])
        for i in range(n_chunks):
            cur = i % 2
            if i + 1 < n_chunks:
                nxt = (i + 1) % 2
                pltpu.sync_copy(idx_hbm.at[wid, i + 1], idxs[nxt])
                cps[nxt] = pltpu.async_copy(
                    table_hbm.at[idxs[nxt]], bufs[nxt], sems[nxt])
            cps[cur].wait()
            pltpu.sync_copy(
                bufs[cur],
                out_hbm.at[pl.ds(base + i * chunk, chunk), pl.ds(0, d)])

    return k(table, idx3)


def _mm_body(emb_ref, wt_ref, b_ref, out_ref):
    _, bs, h = out_ref.shape
    e = emb_ref[:, :64]
    r = jnp.dot(e, wt_ref[...], preferred_element_type=jnp.float32) + b_ref[...]
    out_ref[...] = r.reshape(1, bs, h)


@functools.partial(jax.jit, static_argnames=("seq",))
def _tc_project(emb, wt, b2, *, seq):
    d, h = wt.shape
    n = emb.shape[0]
    bsz = n // seq
    return pl.pallas_call(
        _mm_body,
        grid=(seq,),
        in_specs=[
            pl.BlockSpec((bsz, 128), lambda i: (i, 0)),
            pl.BlockSpec((d, h), lambda i: (0, 0)),
            pl.BlockSpec((1, h), lambda i: (0, 0)),
        ],
        out_specs=pl.BlockSpec((1, bsz, h), lambda i: (i, 0, 0)),
        out_shape=jax.ShapeDtypeStruct((seq, bsz, h), jnp.float32),
    )(emb, wt, b2)


def kernel(x, table, W, b):
    bsz, seq = x.shape
    n_rows = bsz * seq
    chunk = 640
    per_w = n_rows // _NW
    n_chunks = per_w // chunk
    tbl = lax.optimization_barrier(table.reshape(-1)).reshape(table.shape)
    idx3 = x.T.reshape(_NW, n_chunks, chunk).astype(jnp.int32)
    emb = _sc_gather(tbl, idx3, n_chunks=n_chunks, chunk=chunk)
    out = _tc_project(emb, W.T, b.reshape(1, -1), seq=seq)
    return out.transpose(1, 0, 2)


# speedup vs baseline: 2.5172x; 1.2626x over previous
import functools

import jax
import jax.numpy as jnp
from jax import lax
from jax.experimental import pallas as pl
from jax.experimental.pallas import tpu as pltpu
from jax.experimental.pallas import tpu_sc as plsc

_NC = 2
_NS = 16
_NW = _NC * _NS


@functools.partial(jax.jit, static_argnames=("n_chunks", "chunk"))
def _sc_gather(table, idx3, *, n_chunks, chunk):
    n_rows = _NW * n_chunks * chunk
    mesh = plsc.VectorSubcoreMesh(core_axis_name="c", subcore_axis_name="s")

    @functools.partial(
        pl.kernel,
        mesh=mesh,
        out_type=jax.ShapeDtypeStruct((n_rows, 128), jnp.float32),
        scratch_types=[
            pltpu.VMEM((chunk,), jnp.int32),
            pltpu.VMEM((chunk,), jnp.int32),
            pltpu.VMEM((chunk, 128), jnp.float32),
            pltpu.VMEM((chunk, 128), jnp.float32),
            pltpu.SemaphoreType.DMA,
            pltpu.SemaphoreType.DMA,
        ],
        compiler_params=pltpu.CompilerParams(use_tc_tiling_on_sc=False),
    )
    def k(table_hbm, idx_hbm, out_hbm, idx0, idx1, buf0, buf1, sem0, sem1):
        wid = lax.axis_index("s") * _NC + lax.axis_index("c")
        base = wid * (n_chunks * chunk)
        idxs = (idx0, idx1)
        bufs = (buf0, buf1)
        sems = (sem0, sem1)
        cps = [None, None]
        pltpu.sync_copy(idx_hbm.at[wid, 0], idxs[0])
        cps[0] = pltpu.async_copy(table_hbm.at[idxs[0]], bufs[0], sems[0])
        for i in range(n_chunks):
            cur = i % 2
            if i + 1 < n_chunks:
                nxt = (i + 1) % 2
                pltpu.sync_copy(idx_hbm.at[wid, i + 1], idxs[nxt])
                cps[nxt] = pltpu.async_copy(
                    table_hbm.at[idxs[nxt]], bufs[nxt], sems[nxt])
            cps[cur].wait()
            pltpu.sync_copy(bufs[cur], out_hbm.at[pl.ds(base + i * chunk, chunk)])

    return k(table, idx3)


def _tr_body(tt_ref, out_ref):
    t = tt_ref[...].T
    out_ref[:, :64] = t
    out_ref[:, 64:] = t


@jax.jit
def _tc_relayout(table_t):
    v = table_t.shape[1]
    v_pad = ((v + 63) // 64) * 64
    blk = 4096
    grid = ((v_pad + blk - 1) // blk,)
    return pl.pallas_call(
        _tr_body,
        grid=grid,
        in_specs=[pl.BlockSpec((64, blk), lambda i: (0, i))],
        out_specs=pl.BlockSpec((blk, 128), lambda i: (i, 0)),
        out_shape=jax.ShapeDtypeStruct((v_pad, 128), jnp.float32),
    )(table_t)


def _mm_body(emb_ref, wt_ref, b_ref, out_ref):
    _, bs, h = out_ref.shape
    e = emb_ref[:, :64]
    r = jnp.dot(e, wt_ref[...], preferred_element_type=jnp.float32) + b_ref[...]
    out_ref[...] = r.reshape(1, bs, h)


@functools.partial(jax.jit, static_argnames=("seq",))
def _tc_project(emb, wt, b2, *, seq):
    d, h = wt.shape
    n = emb.shape[0]
    bsz = n // seq
    return pl.pallas_call(
        _mm_body,
        grid=(seq,),
        in_specs=[
            pl.BlockSpec((bsz, 128), lambda i: (i, 0)),
            pl.BlockSpec((d, h), lambda i: (0, 0)),
            pl.BlockSpec((1, h), lambda i: (0, 0)),
        ],
        out_specs=pl.BlockSpec((1, bsz, h), lambda i: (i, 0, 0)),
        out_shape=jax.ShapeDtypeStruct((seq, bsz, h), jnp.float32),
    )(emb, wt, b2)


def kernel(x, table, W, b):
    bsz, seq = x.shape
    n_rows = bsz * seq
    chunk = 320
    per_w = n_rows // _NW
    n_chunks = per_w // chunk
    tbl = _tc_relayout(table.T)
    idx3 = x.T.reshape(_NW, n_chunks, chunk).astype(jnp.int32)
    emb = _sc_gather(tbl, idx3, n_chunks=n_chunks, chunk=chunk)
    out = _tc_project(emb, W.T, b.reshape(1, -1), seq=seq)
    return out.transpose(1, 0, 2)


# speedup vs baseline: 2.8035x; 1.1138x over previous
import functools

import jax
import jax.numpy as jnp
from jax import lax
from jax.experimental import pallas as pl
from jax.experimental.pallas import tpu as pltpu
from jax.experimental.pallas import tpu_sc as plsc

_NC = 2
_NS = 16
_NW = _NC * _NS


@functools.partial(jax.jit, static_argnames=("n_chunks", "chunk"))
def _sc_gather(table, idx3, *, n_chunks, chunk):
    n_rows = _NW * n_chunks * chunk
    mesh = plsc.VectorSubcoreMesh(core_axis_name="c", subcore_axis_name="s")

    @functools.partial(
        pl.kernel,
        mesh=mesh,
        out_type=jax.ShapeDtypeStruct((n_rows, 128), jnp.float32),
        scratch_types=[
            pltpu.VMEM((chunk,), jnp.int32),
            pltpu.VMEM((chunk,), jnp.int32),
            pltpu.VMEM((chunk, 128), jnp.float32),
            pltpu.VMEM((chunk, 128), jnp.float32),
            pltpu.SemaphoreType.DMA,
            pltpu.SemaphoreType.DMA,
        ],
        compiler_params=pltpu.CompilerParams(use_tc_tiling_on_sc=False),
    )
    def k(table_hbm, idx_hbm, out_hbm, idx0, idx1, buf0, buf1, sem0, sem1):
        wid = lax.axis_index("s") * _NC + lax.axis_index("c")
        base = wid * (n_chunks * chunk)
        idxs = (idx0, idx1)
        bufs = (buf0, buf1)
        sems = (sem0, sem1)
        cps = [None, None]
        pltpu.sync_copy(idx_hbm.at[wid, 0], idxs[0])
        cps[0] = pltpu.async_copy(table_hbm.at[idxs[0]], bufs[0], sems[0])
        for i in range(n_chunks):
            cur = i % 2
            if i + 1 < n_chunks:
                nxt = (i + 1) % 2
                pltpu.sync_copy(idx_hbm.at[wid, i + 1], idxs[nxt])
                cps[nxt] = pltpu.async_copy(
                    table_hbm.at[idxs[nxt]], bufs[nxt], sems[nxt])
            cps[cur].wait()
            pltpu.sync_copy(
                bufs[cur].at[:, pl.ds(0, 64)],
                out_hbm.at[pl.ds(base + i * chunk, chunk), pl.ds(0, 64)])

    return k(table, idx3)


def _tr_body(tt_ref, out_ref):
    out_ref[:, :64] = tt_ref[...].T


@jax.jit
def _tc_relayout(table_t):
    v = table_t.shape[1]
    v_pad = ((v + 63) // 64) * 64
    blk = 4096
    grid = ((v_pad + blk - 1) // blk,)
    return pl.pallas_call(
        _tr_body,
        grid=grid,
        in_specs=[pl.BlockSpec((64, blk), lambda i: (0, i))],
        out_specs=pl.BlockSpec((blk, 128), lambda i: (i, 0)),
        out_shape=jax.ShapeDtypeStruct((v_pad, 128), jnp.float32),
    )(table_t)


def _mm_body(emb_ref, wt_ref, b_ref, out_ref):
    _, bs, h = out_ref.shape
    e = emb_ref[:, :64]
    r = jnp.dot(e, wt_ref[...], preferred_element_type=jnp.float32) + b_ref[...]
    out_ref[...] = r.reshape(1, bs, h)


@functools.partial(jax.jit, static_argnames=("seq",))
def _tc_project(emb, wt, b2, *, seq):
    d, h = wt.shape
    n = emb.shape[0]
    bsz = n // seq
    return pl.pallas_call(
        _mm_body,
        grid=(seq,),
        in_specs=[
            pl.BlockSpec((bsz, 128), lambda i: (i, 0)),
            pl.BlockSpec((d, h), lambda i: (0, 0)),
            pl.BlockSpec((1, h), lambda i: (0, 0)),
        ],
        out_specs=pl.BlockSpec((1, bsz, h), lambda i: (i, 0, 0)),
        out_shape=jax.ShapeDtypeStruct((seq, bsz, h), jnp.float32),
    )(emb, wt, b2)


def kernel(x, table, W, b):
    bsz, seq = x.shape
    n_rows = bsz * seq
    chunk = 320
    per_w = n_rows // _NW
    n_chunks = per_w // chunk
    tbl = _tc_relayout(table.T)
    idx3 = x.T.reshape(_NW, n_chunks, chunk).astype(jnp.int32)
    emb = _sc_gather(tbl, idx3, n_chunks=n_chunks, chunk=chunk)
    out = _tc_project(emb, W.T, b.reshape(1, -1), seq=seq)
    return out.transpose(1, 0, 2)


# speedup vs baseline: 3.1167x; 1.1117x over previous
import functools

import jax
import jax.numpy as jnp
from jax import lax
from jax.experimental import pallas as pl
from jax.experimental.pallas import tpu as pltpu
from jax.experimental.pallas import tpu_sc as plsc

_NC = 2
_NS = 16
_NW = _NC * _NS
_BLK = 2048
_H = 245 * _BLK


def _tr_body(ta_ref, tb_ref, out_ref):
    out_ref[:, :64] = ta_ref[...].T
    out_ref[:, 64:] = tb_ref[...].T


@jax.jit
def _tc_relayout(table_t):
    return pl.pallas_call(
        _tr_body,
        grid=(_H // _BLK,),
        in_specs=[
            pl.BlockSpec((64, _BLK), lambda i: (0, i)),
            pl.BlockSpec(
                (64, _BLK),
                lambda i: (0, jnp.minimum(i + _H // _BLK, 488))),
        ],
        out_specs=pl.BlockSpec((_BLK, 128), lambda i: (i, 0)),
        out_shape=jax.ShapeDtypeStruct((_H, 128), jnp.float32),
    )(table_t, table_t)


@functools.partial(jax.jit, static_argnames=("n_chunks", "chunk"))
def _sc_gather(table, idx3, *, n_chunks, chunk):
    n_rows = _NW * n_chunks * chunk
    half = n_rows // 2
    mesh = plsc.VectorSubcoreMesh(core_axis_name="c", subcore_axis_name="s")

    @functools.partial(
        pl.kernel,
        mesh=mesh,
        out_type=jax.ShapeDtypeStruct((half, 128), jnp.float32),
        scratch_types=[
            pltpu.VMEM((chunk,), jnp.int32),
            pltpu.VMEM((chunk,), jnp.int32),
            pltpu.VMEM((chunk, 64), jnp.float32),
            pltpu.VMEM((chunk, 64), jnp.float32),
            pltpu.SemaphoreType.DMA,
            pltpu.SemaphoreType.DMA,
        ],
        compiler_params=pltpu.CompilerParams(use_tc_tiling_on_sc=False),
    )
    def k(table_hbm, idx_hbm, out_hbm, idx0, idx1, buf0, buf1, sem0, sem1):
        wid = lax.axis_index("s") * _NC + lax.axis_index("c")
        base = wid * (n_chunks * chunk)
        row0 = lax.rem(base, half)
        col0 = (base // half) * 64
        idxs = (idx0, idx1)
        bufs = (buf0, buf1)
        sems = (sem0, sem1)
        cps = [None, None]
        pltpu.sync_copy(idx_hbm.at[wid, 0], idxs[0])
        cps[0] = pltpu.async_copy(table_hbm.at[idxs[0]], bufs[0], sems[0])
        for i in range(n_chunks):
            cur = i % 2
            if i + 1 < n_chunks:
                nxt = (i + 1) % 2
                pltpu.sync_copy(idx_hbm.at[wid, i + 1], idxs[nxt])
                cps[nxt] = pltpu.async_copy(
                    table_hbm.at[idxs[nxt]], bufs[nxt], sems[nxt])
            cps[cur].wait()
            pltpu.sync_copy(
                bufs[cur],
                out_hbm.at[pl.ds(row0 + i * chunk, chunk), pl.ds(col0, 64)])

    return k(table, idx3)


def _mm_body(emb_ref, wt_ref, b_ref, out_ref):
    e = emb_ref[...]
    wt = wt_ref[...]
    bb = b_ref[...]
    ra = jnp.dot(e[:, :64], wt, preferred_element_type=jnp.float32) + bb
    rb = jnp.dot(e[:, 64:], wt, preferred_element_type=jnp.float32) + bb
    out_ref[0, 0] = ra
    out_ref[1, 0] = rb


@functools.partial(jax.jit, static_argnames=("seq",))
def _tc_project(emb2, wt, b2, *, seq):
    d, h = wt.shape
    bsz = (emb2.shape[0] * 2) // seq
    hs = seq // 2
    return pl.pallas_call(
        _mm_body,
        grid=(hs,),
        in_specs=[
            pl.BlockSpec((bsz, 128), lambda i: (i, 0)),
            pl.BlockSpec((d, h), lambda i: (0, 0)),
            pl.BlockSpec((1, h), lambda i: (0, 0)),
        ],
        out_specs=pl.BlockSpec((2, 1, bsz, h), lambda i: (0, i, 0, 0)),
        out_shape=jax.ShapeDtypeStruct((2, hs, bsz, h), jnp.float32),
    )(emb2, wt, b2)


def kernel(x, table, W, b):
    bsz, seq = x.shape
    n_rows = bsz * seq
    chunk = 640
    per_w = n_rows // _NW
    n_chunks = per_w // chunk
    packed = _tc_relayout(table.T)
    tbl = packed.reshape(-1).reshape(2 * _H, 64)
    xt = x.T.astype(jnp.int32)
    xv = jnp.where(xt < _H, 2 * xt, 2 * (xt - _H) + 1)
    idx3 = xv.reshape(_NW, n_chunks, chunk)
    emb2 = _sc_gather(tbl, idx3, n_chunks=n_chunks, chunk=chunk)
    out4 = _tc_project(emb2, W.T, b.reshape(1, -1), seq=seq)
    return out4.reshape(seq, bsz, -1).transpose(1, 0, 2)


# speedup vs baseline: 3.5688x; 1.1451x over previous
import functools

import jax
import jax.numpy as jnp
from jax import lax
from jax.experimental import pallas as pl
from jax.experimental.pallas import tpu as pltpu
from jax.experimental.pallas import tpu_sc as plsc

_NC = 2
_NS = 16
_NW = _NC * _NS
_BLK = 4096
_H = 123 * _BLK


def _tr_body(ta_ref, tb_ref, out_ref):
    out_ref[:, :64] = ta_ref[...].T
    out_ref[:, 64:] = tb_ref[...].T


@jax.jit
def _tc_relayout(table_t):
    return pl.pallas_call(
        _tr_body,
        grid=(_H // _BLK,),
        in_specs=[
            pl.BlockSpec((64, _BLK), lambda i: (0, i)),
            pl.BlockSpec(
                (64, _BLK),
                lambda i: (0, jnp.minimum(i + _H // _BLK, 244))),
        ],
        out_specs=pl.BlockSpec((_BLK, 128), lambda i: (i, 0)),
        out_shape=jax.ShapeDtypeStruct((_H, 128), jnp.float32),
    )(table_t, table_t)


@functools.partial(jax.jit, static_argnames=("n_chunks", "chunk"))
def _sc_gather(table, idx3, *, n_chunks, chunk):
    n_rows = _NW * n_chunks * chunk
    half = n_rows // 2
    mesh = plsc.VectorSubcoreMesh(core_axis_name="c", subcore_axis_name="s")

    @functools.partial(
        pl.kernel,
        mesh=mesh,
        out_type=jax.ShapeDtypeStruct((half, 128), jnp.float32),
        scratch_types=[
            pltpu.VMEM((chunk,), jnp.int32),
            pltpu.VMEM((chunk,), jnp.int32),
            pltpu.VMEM((chunk, 64), jnp.float32),
            pltpu.VMEM((chunk, 64), jnp.float32),
            pltpu.SemaphoreType.DMA,
            pltpu.SemaphoreType.DMA,
        ],
        compiler_params=pltpu.CompilerParams(use_tc_tiling_on_sc=False),
    )
    def k(table_hbm, idx_hbm, out_hbm, idx0, idx1, buf0, buf1, sem0, sem1):
        wid = lax.axis_index("s") * _NC + lax.axis_index("c")
        base = wid * (n_chunks * chunk)
        row0 = lax.rem(base, half)
        col0 = (base // half) * 64
        idxs = (idx0, idx1)
        bufs = (buf0, buf1)
        sems = (sem0, sem1)
        cps = [None, None]
        pltpu.sync_copy(idx_hbm.at[wid, 0], idxs[0])
        cps[0] = pltpu.async_copy(table_hbm.at[idxs[0]], bufs[0], sems[0])
        for i in range(n_chunks):
            cur = i % 2
            if i + 1 < n_chunks:
                nxt = (i + 1) % 2
                pltpu.sync_copy(idx_hbm.at[wid, i + 1], idxs[nxt])
                cps[nxt] = pltpu.async_copy(
                    table_hbm.at[idxs[nxt]], bufs[nxt], sems[nxt])
            cps[cur].wait()
            pltpu.sync_copy(
                bufs[cur],
                out_hbm.at[pl.ds(row0 + i * chunk, chunk), pl.ds(col0, 64)])

    return k(table, idx3)


def _mm_body(emb_ref, wt_ref, b_ref, out_ref):
    e = emb_ref[...]
    wt = wt_ref[...]
    bb = b_ref[...]
    ra = jnp.dot(e[:, :64], wt, preferred_element_type=jnp.float32) + bb
    rb = jnp.dot(e[:, 64:], wt, preferred_element_type=jnp.float32) + bb
    out_ref[0, 0] = ra
    out_ref[1, 0] = rb


@functools.partial(jax.jit, static_argnames=("seq",))
def _tc_project(emb2, wt, b2, *, seq):
    d, h = wt.shape
    bsz = (emb2.shape[0] * 2) // seq
    hs = seq // 2
    return pl.pallas_call(
        _mm_body,
        grid=(hs,),
        in_specs=[
            pl.BlockSpec((bsz, 128), lambda i: (i, 0)),
            pl.BlockSpec((d, h), lambda i: (0, 0)),
            pl.BlockSpec((1, h), lambda i: (0, 0)),
        ],
        out_specs=pl.BlockSpec((2, 1, bsz, h), lambda i: (0, i, 0, 0)),
        out_shape=jax.ShapeDtypeStruct((2, hs, bsz, h), jnp.float32),
    )(emb2, wt, b2)


def kernel(x, table, W, b):
    bsz, seq = x.shape
    n_rows = bsz * seq
    chunk = 640
    per_w = n_rows // _NW
    n_chunks = per_w // chunk
    packed = _tc_relayout(table.T)
    tbl = packed.reshape(-1).reshape(2 * _H, 64)
    xt = x.T.astype(jnp.int32)
    xv = jnp.where(xt < _H, 2 * xt, 2 * (xt - _H) + 1)
    idx3 = xv.reshape(_NW, n_chunks, chunk)
    emb2 = _sc_gather(tbl, idx3, n_chunks=n_chunks, chunk=chunk)
    out4 = _tc_project(emb2, W.T, b.reshape(1, -1), seq=seq)
    return out4.reshape(seq, bsz, -1).transpose(1, 0, 2)


# speedup vs baseline: 3.8441x; 1.0771x over previous
import functools

import jax
import jax.numpy as jnp
from jax import lax
from jax.experimental import pallas as pl
from jax.experimental.pallas import tpu as pltpu
from jax.experimental.pallas import tpu_sc as plsc

_NC = 2
_NS = 16
_NW = _NC * _NS
_BLK = 8192
_H = 62 * _BLK


def _tr_body(ta_ref, tb_ref, out_ref):
    out_ref[:, :64] = ta_ref[...].T
    out_ref[:, 64:] = tb_ref[...].T


@jax.jit
def _tc_relayout(table_t):
    return pl.pallas_call(
        _tr_body,
        grid=(_H // _BLK,),
        in_specs=[
            pl.BlockSpec((64, _BLK), lambda i: (0, i)),
            pl.BlockSpec(
                (64, _BLK),
                lambda i: (0, jnp.minimum(i + _H // _BLK, 122))),
        ],
        out_specs=pl.BlockSpec((_BLK, 128), lambda i: (i, 0)),
        out_shape=jax.ShapeDtypeStruct((_H, 128), jnp.float32),
    )(table_t, table_t)


@functools.partial(jax.jit, static_argnames=("n_chunks", "chunk"))
def _sc_gather(table, idx3, *, n_chunks, chunk):
    n_rows = _NW * n_chunks * chunk
    half = n_rows // 2
    mesh = plsc.VectorSubcoreMesh(core_axis_name="c", subcore_axis_name="s")

    @functools.partial(
        pl.kernel,
        mesh=mesh,
        out_type=jax.ShapeDtypeStruct((half, 128), jnp.float32),
        scratch_types=[
            pltpu.VMEM((chunk,), jnp.int32),
            pltpu.VMEM((chunk,), jnp.int32),
            pltpu.VMEM((chunk, 64), jnp.float32),
            pltpu.VMEM((chunk, 64), jnp.float32),
            pltpu.SemaphoreType.DMA,
            pltpu.SemaphoreType.DMA,
        ],
        compiler_params=pltpu.CompilerParams(use_tc_tiling_on_sc=False),
    )
    def k(table_hbm, idx_hbm, out_hbm, idx0, idx1, buf0, buf1, sem0, sem1):
        wid = lax.axis_index("s") * _NC + lax.axis_index("c")
        base = wid * (n_chunks * chunk)
        row0 = lax.rem(base, half)
        col0 = (base // half) * 64
        idxs = (idx0, idx1)
        bufs = (buf0, buf1)
        sems = (sem0, sem1)
        cps = [None, None]
        pltpu.sync_copy(idx_hbm.at[wid, 0], idxs[0])
        cps[0] = pltpu.async_copy(table_hbm.at[idxs[0]], bufs[0], sems[0])
        for i in range(n_chunks):
            cur = i % 2
            if i + 1 < n_chunks:
                nxt = (i + 1) % 2
                pltpu.sync_copy(idx_hbm.at[wid, i + 1], idxs[nxt])
                cps[nxt] = pltpu.async_copy(
                    table_hbm.at[idxs[nxt]], bufs[nxt], sems[nxt])
            cps[cur].wait()
            pltpu.sync_copy(
                bufs[cur],
                out_hbm.at[pl.ds(row0 + i * chunk, chunk), pl.ds(col0, 64)])

    return k(table, idx3)


def _mm_body(emb_ref, wt_ref, b_ref, out_ref):
    e = emb_ref[...]
    wt = wt_ref[...]
    bb = b_ref[...]
    ra = jnp.dot(e[:, :64], wt, preferred_element_type=jnp.float32) + bb
    rb = jnp.dot(e[:, 64:], wt, preferred_element_type=jnp.float32) + bb
    out_ref[0, 0] = ra
    out_ref[1, 0] = rb


@functools.partial(jax.jit, static_argnames=("seq",))
def _tc_project(emb2, wt, b2, *, seq):
    d, h = wt.shape
    bsz = (emb2.shape[0] * 2) // seq
    hs = seq // 2
    return pl.pallas_call(
        _mm_body,
        grid=(hs,),
        in_specs=[
            pl.BlockSpec((bsz, 128), lambda i: (i, 0)),
            pl.BlockSpec((d, h), lambda i: (0, 0)),
            pl.BlockSpec((1, h), lambda i: (0, 0)),
        ],
        out_specs=pl.BlockSpec((2, 1, bsz, h), lambda i: (0, i, 0, 0)),
        out_shape=jax.ShapeDtypeStruct((2, hs, bsz, h), jnp.float32),
    )(emb2, wt, b2)


def kernel(x, table, W, b):
    bsz, seq = x.shape
    n_rows = bsz * seq
    chunk = 640
    per_w = n_rows // _NW
    n_chunks = per_w // chunk
    packed = _tc_relayout(table.T)
    tbl = packed.reshape(-1).reshape(2 * _H, 64)
    xt = x.T.astype(jnp.int32)
    xv = jnp.where(xt < _H, 2 * xt, 2 * (xt - _H) + 1)
    idx3 = xv.reshape(_NW, n_chunks, chunk)
    emb2 = _sc_gather(tbl, idx3, n_chunks=n_chunks, chunk=chunk)
    out4 = _tc_project(emb2, W.T, b.reshape(1, -1), seq=seq)
    return out4.reshape(seq, bsz, -1).transpose(1, 0, 2)


# speedup vs baseline: 3.9685x; 1.0324x over previous
import functools

import jax
import jax.numpy as jnp
from jax import lax
from jax.experimental import pallas as pl
from jax.experimental.pallas import tpu as pltpu
from jax.experimental.pallas import tpu_sc as plsc

_NC = 2
_NS = 16
_NW = _NC * _NS
_BLK = 16384
_H = 31 * _BLK


def _tr_body(ta_ref, tb_ref, out_ref):
    out_ref[:, :64] = ta_ref[...].T
    out_ref[:, 64:] = tb_ref[...].T


@jax.jit
def _tc_relayout(table_t):
    return pl.pallas_call(
        _tr_body,
        grid=(_H // _BLK,),
        in_specs=[
            pl.BlockSpec((64, _BLK), lambda i: (0, i)),
            pl.BlockSpec(
                (64, _BLK),
                lambda i: (0, jnp.minimum(i + _H // _BLK, 61))),
        ],
        out_specs=pl.BlockSpec((_BLK, 128), lambda i: (i, 0)),
        out_shape=jax.ShapeDtypeStruct((_H, 128), jnp.float32),
    )(table_t, table_t)


@functools.partial(jax.jit, static_argnames=("n_chunks", "chunk"))
def _sc_gather(table, idx3, *, n_chunks, chunk):
    n_rows = _NW * n_chunks * chunk
    half = n_rows // 2
    mesh = plsc.VectorSubcoreMesh(core_axis_name="c", subcore_axis_name="s")

    @functools.partial(
        pl.kernel,
        mesh=mesh,
        out_type=jax.ShapeDtypeStruct((half, 128), jnp.float32),
        scratch_types=[
            pltpu.VMEM((chunk,), jnp.int32),
            pltpu.VMEM((chunk,), jnp.int32),
            pltpu.VMEM((chunk, 64), jnp.float32),
            pltpu.VMEM((chunk, 64), jnp.float32),
            pltpu.SemaphoreType.DMA,
            pltpu.SemaphoreType.DMA,
        ],
        compiler_params=pltpu.CompilerParams(use_tc_tiling_on_sc=False),
    )
    def k(table_hbm, idx_hbm, out_hbm, idx0, idx1, buf0, buf1, sem0, sem1):
        wid = lax.axis_index("s") * _NC + lax.axis_index("c")
        base = wid * (n_chunks * chunk)
        row0 = lax.rem(base, half)
        col0 = (base // half) * 64
        idxs = (idx0, idx1)
        bufs = (buf0, buf1)
        sems = (sem0, sem1)
        cps = [None, None]
        pltpu.sync_copy(idx_hbm.at[wid, 0], idxs[0])
        cps[0] = pltpu.async_copy(table_hbm.at[idxs[0]], bufs[0], sems[0])
        for i in range(n_chunks):
            cur = i % 2
            if i + 1 < n_chunks:
                nxt = (i + 1) % 2
                pltpu.sync_copy(idx_hbm.at[wid, i + 1], idxs[nxt])
                cps[nxt] = pltpu.async_copy(
                    table_hbm.at[idxs[nxt]], bufs[nxt], sems[nxt])
            cps[cur].wait()
            pltpu.sync_copy(
                bufs[cur],
                out_hbm.at[pl.ds(row0 + i * chunk, chunk), pl.ds(col0, 64)])

    return k(table, idx3)


def _mm_body(emb_ref, wt_ref, b_ref, out_ref):
    e = emb_ref[...]
    wt = wt_ref[...]
    bb = b_ref[...]
    ra = jnp.dot(e[:, :64], wt, preferred_element_type=jnp.float32) + bb
    rb = jnp.dot(e[:, 64:], wt, preferred_element_type=jnp.float32) + bb
    out_ref[0, 0] = ra
    out_ref[1, 0] = rb


@functools.partial(jax.jit, static_argnames=("seq",))
def _tc_project(emb2, wt, b2, *, seq):
    d, h = wt.shape
    bsz = (emb2.shape[0] * 2) // seq
    hs = seq // 2
    return pl.pallas_call(
        _mm_body,
        grid=(hs,),
        in_specs=[
            pl.BlockSpec((bsz, 128), lambda i: (i, 0)),
            pl.BlockSpec((d, h), lambda i: (0, 0)),
            pl.BlockSpec((1, h), lambda i: (0, 0)),
        ],
        out_specs=pl.BlockSpec((2, 1, bsz, h), lambda i: (0, i, 0, 0)),
        out_shape=jax.ShapeDtypeStruct((2, hs, bsz, h), jnp.float32),
    )(emb2, wt, b2)


def kernel(x, table, W, b):
    bsz, seq = x.shape
    n_rows = bsz * seq
    chunk = 640
    per_w = n_rows // _NW
    n_chunks = per_w // chunk
    packed = _tc_relayout(table.T)
    tbl = packed.reshape(-1).reshape(2 * _H, 64)
    xt = x.T.astype(jnp.int32)
    xv = jnp.where(xt < _H, 2 * xt, 2 * (xt - _H) + 1)
    idx3 = xv.reshape(_NW, n_chunks, chunk)
    emb2 = _sc_gather(tbl, idx3, n_chunks=n_chunks, chunk=chunk)
    out4 = _tc_project(emb2, W.T, b.reshape(1, -1), seq=seq)
    return out4.reshape(seq, bsz, -1).transpose(1, 0, 2)


# speedup vs baseline: 3.9816x; 1.0033x over previous
import functools

import jax
import jax.numpy as jnp
from jax import lax
from jax.experimental import pallas as pl
from jax.experimental.pallas import tpu as pltpu
from jax.experimental.pallas import tpu_sc as plsc

_NC = 2
_NS = 16
_NW = _NC * _NS
_BLK = 16384
_H = 31 * _BLK


def _tr_body(ta_ref, tb_ref, out_ref):
    out_ref[:, :64] = ta_ref[...].T
    out_ref[:, 64:] = tb_ref[...].T


@jax.jit
def _tc_relayout(table_t):
    return pl.pallas_call(
        _tr_body,
        grid=(_H // _BLK,),
        in_specs=[
            pl.BlockSpec((64, _BLK), lambda i: (0, i)),
            pl.BlockSpec(
                (64, _BLK),
                lambda i: (0, jnp.minimum(i + _H // _BLK, 61))),
        ],
        out_specs=pl.BlockSpec((_BLK, 128), lambda i: (i, 0)),
        out_shape=jax.ShapeDtypeStruct((_H, 128), jnp.float32),
    )(table_t, table_t)


@functools.partial(jax.jit, static_argnames=("n_chunks", "chunk"))
def _sc_gather(table, idx3, *, n_chunks, chunk):
    n_rows = _NW * n_chunks * chunk
    half = n_rows // 2
    mesh = plsc.VectorSubcoreMesh(core_axis_name="c", subcore_axis_name="s")

    @functools.partial(
        pl.kernel,
        mesh=mesh,
        out_type=jax.ShapeDtypeStruct((half, 128), jnp.float32),
        scratch_types=[
            pltpu.VMEM((chunk,), jnp.int32),
            pltpu.VMEM((chunk,), jnp.int32),
            pltpu.VMEM((chunk, 64), jnp.float32),
            pltpu.VMEM((chunk, 64), jnp.float32),
            pltpu.SemaphoreType.DMA,
            pltpu.SemaphoreType.DMA,
        ],
        compiler_params=pltpu.CompilerParams(use_tc_tiling_on_sc=False),
    )
    def k(table_hbm, idx_hbm, out_hbm, idx0, idx1, buf0, buf1, sem0, sem1):
        wid = lax.axis_index("s") * _NC + lax.axis_index("c")
        base = wid * (n_chunks * chunk)
        row0 = lax.rem(base, half)
        col0 = (base // half) * 64
        idxs = (idx0, idx1)
        bufs = (buf0, buf1)
        sems = (sem0, sem1)
        cps = [None, None]
        pltpu.sync_copy(idx_hbm.at[wid, 0], idxs[0])
        cps[0] = pltpu.async_copy(table_hbm.at[idxs[0]], bufs[0], sems[0])
        for i in range(n_chunks):
            cur = i % 2
            if i + 1 < n_chunks:
                nxt = (i + 1) % 2
                pltpu.sync_copy(idx_hbm.at[wid, i + 1], idxs[nxt])
                cps[nxt] = pltpu.async_copy(
                    table_hbm.at[idxs[nxt]], bufs[nxt], sems[nxt])
            cps[cur].wait()
            pltpu.sync_copy(
                bufs[cur],
                out_hbm.at[pl.ds(row0 + i * chunk, chunk), pl.ds(col0, 64)])

    return k(table, idx3)


def _mm_body(emb_ref, wt_ref, b_ref, out_ref):
    e = emb_ref[...]
    wt = wt_ref[...]
    bb = b_ref[...]
    ra = jnp.dot(e[:, :64], wt, preferred_element_type=jnp.float32) + bb
    rb = jnp.dot(e[:, 64:], wt, preferred_element_type=jnp.float32) + bb
    out_ref[0, 0] = ra
    out_ref[1, 0] = rb


@functools.partial(jax.jit, static_argnames=("seq",))
def _tc_project(emb2, wt, b2, *, seq):
    d, h = wt.shape
    bsz = (emb2.shape[0] * 2) // seq
    hs = seq // 2
    return pl.pallas_call(
        _mm_body,
        grid=(hs,),
        in_specs=[
            pl.BlockSpec((bsz, 128), lambda i: (i, 0)),
            pl.BlockSpec((d, h), lambda i: (0, 0)),
            pl.BlockSpec((1, h), lambda i: (0, 0)),
        ],
        out_specs=pl.BlockSpec((2, 1, bsz, h), lambda i: (0, i, 0, 0)),
        out_shape=jax.ShapeDtypeStruct((2, hs, bsz, h), jnp.float32),
    )(emb2, wt, b2)


def kernel(x, table, W, b):
    bsz, seq = x.shape
    n_rows = bsz * seq
    chunk = 800
    per_w = n_rows // _NW
    n_chunks = per_w // chunk
    packed = _tc_relayout(table.T)
    tbl = packed.reshape(-1).reshape(2 * _H, 64)
    xt = x.T.astype(jnp.int32)
    xv = jnp.where(xt < _H, 2 * xt, 2 * (xt - _H) + 1)
    idx3 = xv.reshape(_NW, n_chunks, chunk)
    emb2 = _sc_gather(tbl, idx3, n_chunks=n_chunks, chunk=chunk)
    out4 = _tc_project(emb2, W.T, b.reshape(1, -1), seq=seq)
    return out4.reshape(seq, bsz, -1).transpose(1, 0, 2)
